# Initial kernel scaffold; baseline (speedup 1.0000x reference)
import jax
import jax.numpy as jnp
from jax.experimental import pallas as pl


def kernel(h, Wq, bq, Wk, bk, Wv, bv, Wpk, bpk, Wpv, bpv):
    raise NotImplementedError("write your pallas kernel here")


# speedup vs baseline: 8.2575x; 8.2575x over previous
import jax
import jax.numpy as jnp
from jax.experimental import pallas as pl

_B, _S, _D = 2, 2048, 1024
_C = 64
_K = 8
_LB = 128
_NEG = -1e9
_BT = 256
_W = _S // _LB + 1
_SCALE = 1.0 / 32.0
_HI = jax.lax.Precision.HIGHEST


def _proj_body(h_ref, w_ref, b_ref, o_ref):
    j = pl.program_id(0)
    r = pl.program_id(1)
    s = r % _W

    @pl.when(s == 0)
    def _zero():
        o_ref[...] = jnp.zeros_like(o_ref)

    @pl.when((s != 0) & (j < 3))
    def _plain():
        y = jax.lax.dot_general(h_ref[...].astype(jnp.bfloat16),
                                w_ref[0].astype(jnp.bfloat16),
                                (((1,), (1,)), ((), ())),
                                preferred_element_type=jnp.float32)
        o_ref[0] = y + b_ref[0]

    @pl.when((s != 0) & (j >= 3))
    def _phrase():
        x = h_ref[...]
        rr = jax.lax.broadcasted_iota(jnp.int32, (_LB, _LB), 0)
        cc = jax.lax.broadcasted_iota(jnp.int32, (_LB, _LB), 1)
        tri = ((rr // _C) == (cc // _C)) & (cc <= rr)
        tmat = jnp.where(tri, 1.0 / (1.0 + (rr % _C).astype(jnp.float32)), 0.0)
        ps = jax.lax.dot_general(tmat, x, (((1,), (0,)), ((), ())),
                                 preferred_element_type=jnp.float32, precision=_HI)
        y = jax.lax.dot_general(ps.astype(jnp.bfloat16),
                                w_ref[0].astype(jnp.bfloat16),
                                (((1,), (1,)), ((), ())),
                                preferred_element_type=jnp.float32)
        o_ref[0] = y + b_ref[0]


def _attn_body(q1, q2, k0, k1, k2, v0, v1, v2,
               pk0, pk1, pk2, pv0, pv1, pv2, o_ref):
    i = pl.program_id(1)
    t0 = i * _BT
    q = jnp.concatenate([q1[0], q2[0]], axis=0)
    kw = jnp.concatenate([k0[0], k1[0], k2[0]], axis=0)
    vw = jnp.concatenate([v0[0], v1[0], v2[0]], axis=0)
    pkw = jnp.concatenate([pk0[0], pk1[0], pk2[0]], axis=0)
    pvw = jnp.concatenate([pv0[0], pv1[0], pv2[0]], axis=0)

    scores = jax.lax.dot_general(q.astype(jnp.bfloat16),
                                 kw.astype(jnp.bfloat16),
                                 (((1,), (1,)), ((), ())),
                                 preferred_element_type=jnp.float32) * _SCALE
    ii = jax.lax.broadcasted_iota(jnp.int32, (_BT, _BT + _LB), 0)
    jj = jax.lax.broadcasted_iota(jnp.int32, (_BT, _BT + _LB), 1)
    valid = (jj >= ii) & (jj < ii + _LB) & (jj + t0 >= _LB)
    masked = jnp.where(valid, scores, _NEG)

    cur = masked
    keep = jnp.zeros(masked.shape, dtype=jnp.bool_)
    m1 = None
    for it in range(_K):
        m = jnp.max(cur, axis=1, keepdims=True)
        if it == 0:
            m1 = m[:, 0]
        am = jnp.min(jnp.where(cur == m, jj, 2 ** 30), axis=1, keepdims=True)
        knock = jj == am
        keep = keep | knock
        cur = jnp.where(knock, _NEG, cur)

    kprev = kw[_LB - 1:_LB - 1 + _BT]
    pkprev = pkw[_LB - 1:_LB - 1 + _BT]
    seq = jnp.sum(q * kprev, axis=1) * _SCALE
    ph = jnp.sum(q * pkprev, axis=1) * _SCALE

    m10 = jnp.maximum(jnp.maximum(seq, ph), m1)
    eb = jnp.where(keep, jnp.exp(masked - m10[:, None]), 0.0)
    es = jnp.exp(seq - m10)
    ep = jnp.exp(ph - m10)
    z = es + ep + jnp.sum(eb, axis=1)

    acc = jax.lax.dot_general(eb, vw, (((1,), (0,)), ((), ())),
                              preferred_element_type=jnp.float32, precision=_HI)
    acc = acc + es[:, None] * vw[_LB - 1:_LB - 1 + _BT]
    acc = acc + ep[:, None] * pvw[_LB - 1:_LB - 1 + _BT]
    out = acc / z[:, None]

    tvec = t0 + jax.lax.broadcasted_iota(jnp.int32, (_BT, 1), 0)
    o_ref[0] = jnp.where(tvec > 0, out, 0.0)


def kernel(h, Wq, bq, Wk, bk, Wv, bv, Wpk, bpk, Wpv, bpv):
    hf = h.reshape(_B * _S, _D)
    wall = jnp.stack([Wq, Wk, Wv, Wpk, Wpv])
    ball = jnp.stack([bq, bk, bv, bpk, bpv]).reshape(5, 1, _D)

    proj = pl.pallas_call(
        _proj_body,
        grid=(5, _B * _W),
        in_specs=[
            pl.BlockSpec((_LB, _D),
                         lambda j, r: ((r // _W) * (_S // _LB)
                                       + jnp.maximum(r % _W - 1, 0), 0)),
            pl.BlockSpec((1, _D, _D), lambda j, r: (j, 0, 0)),
            pl.BlockSpec((1, 1, _D), lambda j, r: (j, 0, 0)),
        ],
        out_specs=pl.BlockSpec((1, _LB, _D),
                               lambda j, r: (j * _B + r // _W, r % _W, 0)),
        out_shape=jax.ShapeDtypeStruct((5 * _B, _S + _LB, _D), jnp.float32),
    )(hf, wall, ball)

    def wspec(jplane, t):
        return pl.BlockSpec(
            (1, _LB, _D),
            lambda b, i, jplane=jplane, t=t: (jplane * _B + b, 2 * i + t, 0))

    in_specs = [wspec(0, 1), wspec(0, 2)]
    for jplane in (1, 2, 3, 4):
        in_specs += [wspec(jplane, 0), wspec(jplane, 1), wspec(jplane, 2)]

    out = pl.pallas_call(
        _attn_body,
        grid=(_B, _S // _BT),
        in_specs=in_specs,
        out_specs=pl.BlockSpec((1, _BT, _D), lambda b, i: (b, i, 0)),
        out_shape=jax.ShapeDtypeStruct((_B, _S, _D), jnp.float32),
    )(*([proj] * 14))
    return out


# speedup vs baseline: 16.8192x; 2.0368x over previous
import jax
import jax.numpy as jnp
from jax.experimental import pallas as pl
from jax.experimental.pallas import tpu as pltpu

_B, _S, _D = 2, 2048, 1024
_C = 64
_K = 8
_LB = 128
_NEG = -1e9
_BT = 256
_NB = _S // _BT
_SCALE = 1.0 / 32.0
_HI = jax.lax.Precision.HIGHEST


def _bdot(a_bf, b_bf):
    return jax.lax.dot_general(a_bf, b_bf, (((1,), (1,)), ((), ())),
                               preferred_element_type=jnp.float32)


def _fused_body(h_ref, w_ref, b_ref, o_ref, kc, vc, pkc, pvc):
    i = pl.program_id(1)
    t0 = i * _BT

    @pl.when(i == 0)
    def _reset():
        kc[...] = jnp.zeros_like(kc)
        vc[...] = jnp.zeros_like(vc)
        pkc[...] = jnp.zeros_like(pkc)
        pvc[...] = jnp.zeros_like(pvc)

    x = h_ref[...]
    xb = x.astype(jnp.bfloat16)
    q = _bdot(xb, w_ref[0]) + b_ref[0]
    k = _bdot(xb, w_ref[1]) + b_ref[1]
    v = _bdot(xb, w_ref[2]) + b_ref[2]

    rr = jax.lax.broadcasted_iota(jnp.int32, (_BT, _BT), 0)
    cc = jax.lax.broadcasted_iota(jnp.int32, (_BT, _BT), 1)
    tri = ((rr // _C) == (cc // _C)) & (cc <= rr)
    tmat = jnp.where(tri, 1.0 / (1.0 + (rr % _C).astype(jnp.float32)), 0.0)
    ps = jax.lax.dot_general(tmat, x, (((1,), (0,)), ((), ())),
                             preferred_element_type=jnp.float32, precision=_HI)
    psb = ps.astype(jnp.bfloat16)
    pk = _bdot(psb, w_ref[3]) + b_ref[3]
    pv = _bdot(psb, w_ref[4]) + b_ref[4]

    kw = jnp.concatenate([kc[...], k], axis=0)
    vw = jnp.concatenate([vc[...], v], axis=0)
    pkw = jnp.concatenate([pkc[...], pk], axis=0)
    pvw = jnp.concatenate([pvc[...], pv], axis=0)

    scores = _bdot(q.astype(jnp.bfloat16),
                   kw.astype(jnp.bfloat16)) * _SCALE
    ii = jax.lax.broadcasted_iota(jnp.int32, (_BT, _BT + _LB), 0)
    jj = jax.lax.broadcasted_iota(jnp.int32, (_BT, _BT + _LB), 1)
    valid = (jj >= ii) & (jj < ii + _LB) & (jj + t0 >= _LB)
    masked = jnp.where(valid, scores, _NEG)

    cur = masked
    keep = jnp.zeros(masked.shape, dtype=jnp.bool_)
    m1 = None
    for it in range(_K):
        m = jnp.max(cur, axis=1, keepdims=True)
        if it == 0:
            m1 = m[:, 0]
        am = jnp.min(jnp.where(cur == m, jj, 2 ** 30), axis=1, keepdims=True)
        knock = jj == am
        keep = keep | knock
        cur = jnp.where(knock, _NEG, cur)

    kprev = kw[_LB - 1:_LB - 1 + _BT]
    pkprev = pkw[_LB - 1:_LB - 1 + _BT]
    seq = jnp.sum(q * kprev, axis=1) * _SCALE
    ph = jnp.sum(q * pkprev, axis=1) * _SCALE

    m10 = jnp.maximum(jnp.maximum(seq, ph), m1)
    eb = jnp.where(keep, jnp.exp(masked - m10[:, None]), 0.0)
    es = jnp.exp(seq - m10)
    ep = jnp.exp(ph - m10)
    z = es + ep + jnp.sum(eb, axis=1)

    acc = jax.lax.dot_general(eb, vw, (((1,), (0,)), ((), ())),
                              preferred_element_type=jnp.float32, precision=_HI)
    acc = acc + es[:, None] * vw[_LB - 1:_LB - 1 + _BT]
    acc = acc + ep[:, None] * pvw[_LB - 1:_LB - 1 + _BT]
    out = acc / z[:, None]

    tvec = t0 + jax.lax.broadcasted_iota(jnp.int32, (_BT, 1), 0)
    o_ref[0] = jnp.where(tvec > 0, out, 0.0)

    kc[...] = k[_BT - _LB:]
    vc[...] = v[_BT - _LB:]
    pkc[...] = pk[_BT - _LB:]
    pvc[...] = pv[_BT - _LB:]


def kernel(h, Wq, bq, Wk, bk, Wv, bv, Wpk, bpk, Wpv, bpv):
    hf = h.reshape(_B * _S, _D)
    wall = jnp.stack([Wq, Wk, Wv, Wpk, Wpv]).astype(jnp.bfloat16)
    ball = jnp.stack([bq, bk, bv, bpk, bpv]).reshape(5, 1, _D)

    out = pl.pallas_call(
        _fused_body,
        grid=(_B, _NB),
        in_specs=[
            pl.BlockSpec((_BT, _D), lambda b, i: (b * _NB + i, 0)),
            pl.BlockSpec((5, _D, _D), lambda b, i: (0, 0, 0)),
            pl.BlockSpec((5, 1, _D), lambda b, i: (0, 0, 0)),
        ],
        out_specs=pl.BlockSpec((1, _BT, _D), lambda b, i: (b, i, 0)),
        out_shape=jax.ShapeDtypeStruct((_B, _S, _D), jnp.float32),
        scratch_shapes=[pltpu.VMEM((_LB, _D), jnp.float32)] * 4,
    )(hf, wall, ball)
    return out


# speedup vs baseline: 19.5576x; 1.1628x over previous
import jax
import jax.numpy as jnp
from jax.experimental import pallas as pl
from jax.experimental.pallas import tpu as pltpu

_B, _S, _D = 2, 2048, 1024
_C = 64
_K = 8
_LB = 128
_NEG = -1e9
_BT = 256
_NB = _S // _BT
_SCALE = 1.0 / 32.0
_HI = jax.lax.Precision.HIGHEST


def _bdot(a_bf, b_bf):
    return jax.lax.dot_general(a_bf, b_bf, (((1,), (1,)), ((), ())),
                               preferred_element_type=jnp.float32)


def _fused_body(h_ref, w_ref, b_ref, o_ref, kc, vc, pkc, pvc):
    i = pl.program_id(1)
    t0 = i * _BT

    @pl.when(i == 0)
    def _reset():
        kc[...] = jnp.zeros_like(kc)
        vc[...] = jnp.zeros_like(vc)
        pkc[...] = jnp.zeros_like(pkc)
        pvc[...] = jnp.zeros_like(pvc)

    x = h_ref[...]
    xb = x.astype(jnp.bfloat16)
    q = _bdot(xb, w_ref[0]) + b_ref[0]
    k = _bdot(xb, w_ref[1]) + b_ref[1]
    v = _bdot(xb, w_ref[2]) + b_ref[2]

    rr = jax.lax.broadcasted_iota(jnp.int32, (_BT, _BT), 0)
    cc = jax.lax.broadcasted_iota(jnp.int32, (_BT, _BT), 1)
    tri = ((rr // _C) == (cc // _C)) & (cc <= rr)
    tmat = jnp.where(tri, 1.0 / (1.0 + (rr % _C).astype(jnp.float32)), 0.0)
    ps = jax.lax.dot_general(tmat, x, (((1,), (0,)), ((), ())),
                             preferred_element_type=jnp.float32, precision=_HI)
    psb = ps.astype(jnp.bfloat16)
    pk = _bdot(psb, w_ref[3]) + b_ref[3]
    pv = _bdot(psb, w_ref[4]) + b_ref[4]

    kw = jnp.concatenate([kc[...], k], axis=0)
    vw = jnp.concatenate([vc[...], v], axis=0)
    pkw = jnp.concatenate([pkc[...], pk], axis=0)
    pvw = jnp.concatenate([pvc[...], pv], axis=0)
    qb = q.astype(jnp.bfloat16)
    kwb = kw.astype(jnp.bfloat16)

    for half in range(2):
        r0 = half * _LB
        qh = q[r0:r0 + _LB]
        kw_h = kw[r0:r0 + 2 * _LB]
        vw_h = vw[r0:r0 + 2 * _LB]
        pkw_h = pkw[r0:r0 + 2 * _LB]
        pvw_h = pvw[r0:r0 + 2 * _LB]

        scores = jax.lax.dot_general(
            qb[r0:r0 + _LB], kwb[r0:r0 + 2 * _LB], (((1,), (1,)), ((), ())),
            preferred_element_type=jnp.float32) * _SCALE
        ii = jax.lax.broadcasted_iota(jnp.int32, (_LB, 2 * _LB), 0)
        jj = jax.lax.broadcasted_iota(jnp.int32, (_LB, 2 * _LB), 1)
        valid = (jj >= ii) & (jj < ii + _LB) & (jj + r0 + t0 >= _LB)
        masked = jnp.where(valid, scores, _NEG)

        cur = masked
        m1 = None
        for it in range(_K):
            m = jnp.max(cur, axis=1, keepdims=True)
            if it == 0:
                m1 = m[:, 0]
            cur = jnp.where(cur == m, _NEG, cur)
            if it == _K - 1:
                thresh = m[:, 0]
        keep = masked >= thresh[:, None]

        kprev = kw_h[_LB - 1:2 * _LB - 1]
        pkprev = pkw_h[_LB - 1:2 * _LB - 1]
        seq = jnp.sum(qh * kprev, axis=1) * _SCALE
        ph = jnp.sum(qh * pkprev, axis=1) * _SCALE

        m10 = jnp.maximum(jnp.maximum(seq, ph), m1)
        eb = jnp.where(keep, jnp.exp(masked - m10[:, None]), 0.0)
        es = jnp.exp(seq - m10)
        ep = jnp.exp(ph - m10)
        z = es + ep + jnp.sum(eb, axis=1)

        acc = jax.lax.dot_general(eb, vw_h, (((1,), (0,)), ((), ())),
                                  preferred_element_type=jnp.float32,
                                  precision=_HI)
        acc = acc + es[:, None] * vw_h[_LB - 1:2 * _LB - 1]
        acc = acc + ep[:, None] * pvw_h[_LB - 1:2 * _LB - 1]
        out = acc / z[:, None]

        tvec = t0 + r0 + jax.lax.broadcasted_iota(jnp.int32, (_LB, 1), 0)
        o_ref[0, r0:r0 + _LB] = jnp.where(tvec > 0, out, 0.0)

    kc[...] = k[_BT - _LB:]
    vc[...] = v[_BT - _LB:]
    pkc[...] = pk[_BT - _LB:]
    pvc[...] = pv[_BT - _LB:]


def kernel(h, Wq, bq, Wk, bk, Wv, bv, Wpk, bpk, Wpv, bpv):
    hf = h.reshape(_B * _S, _D)
    wall = jnp.stack([Wq, Wk, Wv, Wpk, Wpv]).astype(jnp.bfloat16)
    ball = jnp.stack([bq, bk, bv, bpk, bpv]).reshape(5, 1, _D)

    out = pl.pallas_call(
        _fused_body,
        grid=(_B, _NB),
        in_specs=[
            pl.BlockSpec((_BT, _D), lambda b, i: (b * _NB + i, 0)),
            pl.BlockSpec((5, _D, _D), lambda b, i: (0, 0, 0)),
            pl.BlockSpec((5, 1, _D), lambda b, i: (0, 0, 0)),
        ],
        out_specs=pl.BlockSpec((1, _BT, _D), lambda b, i: (b, i, 0)),
        out_shape=jax.ShapeDtypeStruct((_B, _S, _D), jnp.float32),
        scratch_shapes=[pltpu.VMEM((_LB, _D), jnp.float32)] * 4,
    )(hf, wall, ball)
    return out


# speedup vs baseline: 20.6874x; 1.0578x over previous
import jax
import jax.numpy as jnp
from jax.experimental import pallas as pl
from jax.experimental.pallas import tpu as pltpu

_B, _S, _D = 2, 2048, 1024
_C = 64
_K = 8
_LB = 128
_NEG = -1e9
_BT = 256
_NB = _S // _BT
_SCALE = 1.0 / 32.0
_HI = jax.lax.Precision.HIGHEST


def _bdot(a_bf, b_bf):
    return jax.lax.dot_general(a_bf, b_bf, (((1,), (1,)), ((), ())),
                               preferred_element_type=jnp.float32)


def _fused_body(h_ref, w_ref, b_ref, o_ref, kc, vc, pkc, pvc):
    i = pl.program_id(1)
    t0 = i * _BT

    @pl.when(i == 0)
    def _reset():
        kc[...] = jnp.zeros_like(kc)
        vc[...] = jnp.zeros_like(vc)
        pkc[...] = jnp.zeros_like(pkc)
        pvc[...] = jnp.zeros_like(pvc)

    x = h_ref[...]
    xb = x.astype(jnp.bfloat16)
    q = _bdot(xb, w_ref[0]) + b_ref[0]
    k = _bdot(xb, w_ref[1]) + b_ref[1]
    v = _bdot(xb, w_ref[2]) + b_ref[2]

    rr = jax.lax.broadcasted_iota(jnp.int32, (_BT, _BT), 0)
    cc = jax.lax.broadcasted_iota(jnp.int32, (_BT, _BT), 1)
    tri = ((rr // _C) == (cc // _C)) & (cc <= rr)
    tmat = jnp.where(tri, 1.0 / (1.0 + (rr % _C).astype(jnp.float32)), 0.0)
    ps = jax.lax.dot_general(tmat, x, (((1,), (0,)), ((), ())),
                             preferred_element_type=jnp.float32, precision=_HI)
    psb = ps.astype(jnp.bfloat16)
    pk = _bdot(psb, w_ref[3]) + b_ref[3]
    pv = _bdot(psb, w_ref[4]) + b_ref[4]

    kw = jnp.concatenate([kc[...], k], axis=0)
    vw = jnp.concatenate([vc[...], v], axis=0)
    pkw = jnp.concatenate([pkc[...], pk], axis=0)
    pvw = jnp.concatenate([pvc[...], pv], axis=0)
    qb = q.astype(jnp.bfloat16)
    kwb = kw.astype(jnp.bfloat16)

    for half in range(2):
        r0 = half * _LB
        qh = q[r0:r0 + _LB]
        kw_h = kw[r0:r0 + 2 * _LB]
        vw_h = vw[r0:r0 + 2 * _LB]
        pkw_h = pkw[r0:r0 + 2 * _LB]
        pvw_h = pvw[r0:r0 + 2 * _LB]

        scores = jax.lax.dot_general(
            qb[r0:r0 + _LB], kwb[r0:r0 + 2 * _LB], (((1,), (1,)), ((), ())),
            preferred_element_type=jnp.float32) * _SCALE
        ii = jax.lax.broadcasted_iota(jnp.int32, (_LB, 2 * _LB), 0)
        jj = jax.lax.broadcasted_iota(jnp.int32, (_LB, 2 * _LB), 1)
        valid = (jj >= ii) & (jj < ii + _LB) & (jj + r0 + t0 >= _LB)
        masked = jnp.where(valid, scores, _NEG)

        cur = masked
        m1 = None
        for it in range(_K):
            m = jnp.max(cur, axis=1, keepdims=True)
            if it == 0:
                m1 = m[:, 0]
            cur = jnp.where(cur == m, _NEG, cur)
            if it == _K - 1:
                thresh = m[:, 0]
        keep = masked >= thresh[:, None]

        kprev = kw_h[_LB - 1:2 * _LB - 1]
        pkprev = pkw_h[_LB - 1:2 * _LB - 1]
        seq = jnp.sum(qh * kprev, axis=1) * _SCALE
        ph = jnp.sum(qh * pkprev, axis=1) * _SCALE

        m10 = jnp.maximum(jnp.maximum(seq, ph), m1)
        eb = jnp.where(keep, jnp.exp(masked - m10[:, None]), 0.0)
        es = jnp.exp(seq - m10)
        ep = jnp.exp(ph - m10)
        z = es + ep + jnp.sum(eb, axis=1)

        acc = jax.lax.dot_general(eb.astype(jnp.bfloat16),
                                  vw_h.astype(jnp.bfloat16),
                                  (((1,), (0,)), ((), ())),
                                  preferred_element_type=jnp.float32)
        acc = acc + es[:, None] * vw_h[_LB - 1:2 * _LB - 1]
        acc = acc + ep[:, None] * pvw_h[_LB - 1:2 * _LB - 1]
        out = acc / z[:, None]

        tvec = t0 + r0 + jax.lax.broadcasted_iota(jnp.int32, (_LB, 1), 0)
        o_ref[0, r0:r0 + _LB] = jnp.where(tvec > 0, out, 0.0)

    kc[...] = k[_BT - _LB:]
    vc[...] = v[_BT - _LB:]
    pkc[...] = pk[_BT - _LB:]
    pvc[...] = pv[_BT - _LB:]


def kernel(h, Wq, bq, Wk, bk, Wv, bv, Wpk, bpk, Wpv, bpv):
    hf = h.reshape(_B * _S, _D)
    wall = jnp.stack([Wq, Wk, Wv, Wpk, Wpv]).astype(jnp.bfloat16)
    ball = jnp.stack([bq, bk, bv, bpk, bpv]).reshape(5, 1, _D)

    out = pl.pallas_call(
        _fused_body,
        grid=(_B, _NB),
        in_specs=[
            pl.BlockSpec((_BT, _D), lambda b, i: (b * _NB + i, 0)),
            pl.BlockSpec((5, _D, _D), lambda b, i: (0, 0, 0)),
            pl.BlockSpec((5, 1, _D), lambda b, i: (0, 0, 0)),
        ],
        out_specs=pl.BlockSpec((1, _BT, _D), lambda b, i: (b, i, 0)),
        out_shape=jax.ShapeDtypeStruct((_B, _S, _D), jnp.float32),
        scratch_shapes=[pltpu.VMEM((_LB, _D), jnp.float32)] * 4,
    )(hf, wall, ball)
    return out
